# baseline (speedup 1.0000x reference)
import functools

import jax
import jax.numpy as jnp
from jax import lax
from jax.experimental import pallas as pl
from jax.experimental.pallas import tpu as pltpu
from jax.experimental.pallas import tpu_sc as plsc

V = 1_000_000
N = 16_384
E = 16_384
L = 16
NBLK = V // 128
TAIL0 = NBLK * 128
BPS = 489
WPAIRS = 6
SEG = 512
STG = 64
DUMMY = N

_params = pltpu.CompilerParams(
    use_tc_tiling_on_sc=True, needs_layout_passes=False)
_mesh = lambda: plsc.VectorSubcoreMesh(core_axis_name="c", subcore_axis_name="s")


def _scalar(v):
    return v[0]


@functools.lru_cache(maxsize=None)
def _build_extract():
    emb = jax.ShapeDtypeStruct((N + 1, 128), jnp.float32)

    @functools.partial(
        pl.kernel,
        mesh=_mesh(),
        out_type=(emb, emb, emb, emb),
        scratch_types=[
            pltpu.VMEM((1024,), jnp.int32),
            pltpu.VMEM((16400,), jnp.int32),
            pltpu.VMEM((192, 128), jnp.float32),
            pltpu.VMEM((528,), jnp.int32),
            pltpu.VMEM((STG, 128), jnp.float32),
            pltpu.VMEM((STG,), jnp.int32),
            pltpu.VMEM((2048,), jnp.float32),
            pltpu.SemaphoreType.DMA,
            pltpu.SemaphoreType.DMA,
        ],
        compiler_params=_params,
    )
    def extract(ttu, tti, tlu, tli, uidx, iidx,
                u0_out, u1_out, i0_out, i1_out,
                bufa, nlist, win, matchbuf, stg_rows, stg_nodes, tailbuf,
                sem_w, sem_s):
        core = lax.axis_index("c")
        sub = lax.axis_index("s")
        iota = lax.iota(jnp.int32, L)
        lo = sub * BPS
        ncols = jnp.minimum(jnp.minimum(lo + BPS, NBLK + 1) * 128, V) - lo * 128
        nstream = jnp.minimum(BPS, NBLK - lo)
        nwin = (nstream + WPAIRS - 1) // WPAIRS
        dt0 = 2 * core

        def scan_table(idx_hbm, list_ref):
            def chunk_body(k, cnt):
                pltpu.sync_copy(idx_hbm.at[pl.ds(k * 1024, 1024)], bufa)
                def vreg_body(v, cnt):
                    g = bufa[pl.ds(v * L, L)]
                    lcol = g - lo * 128
                    m = (lcol >= 0) & (lcol < ncols)
                    node = k * 1024 + v * L + iota
                    packed = (lcol << 15) | node
                    plsc.store_compressed(
                        list_ref.at[pl.ds(cnt, L)], packed, mask=m)
                    pc = _scalar(plsc.all_reduce_population_count(m))
                    return cnt + pc
                return lax.fori_loop(0, 64, vreg_body, cnt)
            return lax.fori_loop(0, N // 1024, chunk_body, jnp.int32(0))

        pltpu.sync_copy(tlu.at[pl.ds(core * 1024, 1024)],
                        tailbuf.at[pl.ds(0, 1024)])
        pltpu.sync_copy(tli.at[pl.ds(core * 1024, 1024)],
                        tailbuf.at[pl.ds(1024, 1024)])

        def fetch_window(tt, w, slotbase):
            for j in range(WPAIRS):
                r = (lo + jnp.minimum(w * WPAIRS + j, nstream - 1)) * 128
                pltpu.async_copy(tt.at[dt0, :, pl.ds(r, 128)],
                                 win.at[pl.ds(slotbase + j * L, 8)], sem_w)
                pltpu.async_copy(tt.at[dt0 + 1, :, pl.ds(r, 128)],
                                 win.at[pl.ds(slotbase + j * L + 8, 8)], sem_w)

        def drain_window():
            for j in range(2 * WPAIRS):
                pltpu.make_async_copy(
                    ttu.at[0, :, pl.ds(0, 128)],
                    win.at[pl.ds(j * 8, 8)], sem_w).wait()

        def inject_tail(w, slotbase, tail_off):
            @pl.when((sub == 15) & (w == nwin - 1))
            def _inject():
                j = nstream - (nwin - 1) * WPAIRS
                for d in range(L):
                    for q in range(4):
                        win[slotbase + j * L + d, pl.ds(q * L, L)] = tailbuf[
                            pl.ds(tail_off + d * 64 + q * L, L)]

        def flush_staging(emb0, emb1, scnt):
            for q in range(STG // L):
                idxv = q * L + iota
                cur = stg_nodes[pl.ds(q * L, L)]
                stg_nodes[pl.ds(q * L, L)] = jnp.where(
                    idxv < scnt, cur, jnp.int32(DUMMY))

            @pl.when(core == 0)
            def _c0():
                pltpu.async_copy(stg_rows, emb0.at[stg_nodes], sem_s).wait()

            @pl.when(core == 1)
            def _c1():
                pltpu.async_copy(stg_rows, emb1.at[stg_nodes], sem_s).wait()

        def process_window(list_ref, cnt, emb0, emb1, w, slotbase):
            wlo = w * (WPAIRS * 128)
            whi = jnp.minimum(wlo + WPAIRS * 128, ncols)
            nseg = (cnt + SEG - 1) // SEG

            def seg_body(s_i, scnt):
                def match_body(v, mcnt):
                    eidx = s_i * SEG + v * L
                    p = list_ref[pl.ds(eidx, L)]
                    lcol = p >> 15
                    m = ((lcol >= wlo) & (lcol < whi)
                         & ((eidx + iota) < cnt))
                    plsc.store_compressed(
                        matchbuf.at[pl.ds(mcnt, L)], p, mask=m)
                    return mcnt + _scalar(plsc.all_reduce_population_count(m))
                mcnt = lax.fori_loop(0, SEG // L, match_body, jnp.int32(0))
                matchbuf[pl.ds(mcnt, L)] = jnp.full(
                    (L,), jnp.int32(DUMMY), jnp.int32) | (wlo << 15)

                def group_body(g, scnt):
                    pv = matchbuf[pl.ds(g * L, L)]
                    node = pv & 0x7FFF
                    lcol = pv >> 15
                    rows = slotbase + ((lcol >> 7) - w * WPAIRS) * L
                    cc = lcol & 127
                    for d in range(L):
                        vals = plsc.load_gather(win, [rows + d, cc])
                        plsc.store_scatter(
                            stg_rows,
                            [scnt + iota, jnp.full((L,), d, jnp.int32)], vals)
                    stg_nodes[pl.ds(scnt, L)] = node
                    scnt = scnt + L

                    @pl.when(scnt == STG)
                    def _flush():
                        @pl.when(core == 0)
                        def _c0():
                            pltpu.async_copy(
                                stg_rows, emb0.at[stg_nodes], sem_s).wait()

                        @pl.when(core == 1)
                        def _c1():
                            pltpu.async_copy(
                                stg_rows, emb1.at[stg_nodes], sem_s).wait()
                    return jnp.where(scnt == STG, 0, scnt)

                return lax.fori_loop(0, (mcnt + L - 1) // L, group_body, scnt)

            scnt = lax.fori_loop(0, nseg, seg_body, jnp.int32(0))
            flush_staging(emb0, emb1, scnt)

        def stream_table(tt, list_ref, cnt, emb0, emb1, tail_off):
            fetch_window(tt, jnp.int32(0), 0)

            def win_body(w, carry):
                slot = (w % 2) * (WPAIRS * L)
                nslot = ((w + 1) % 2) * (WPAIRS * L)

                @pl.when(w + 1 < nwin)
                def _prefetch():
                    fetch_window(tt, w + 1, nslot)
                drain_window()
                inject_tail(w, slot, tail_off)
                process_window(list_ref, cnt, emb0, emb1, w, slot)
                return carry
            lax.fori_loop(0, nwin, win_body, jnp.int32(0))

        ucnt = scan_table(uidx, nlist)
        stream_table(ttu, nlist, ucnt, u0_out, u1_out, 0)
        icnt = scan_table(iidx, nlist)
        stream_table(tti, nlist, icnt, i0_out, i1_out, 1024)

    return extract


@functools.lru_cache(maxsize=None)
def _build_score():
    @functools.partial(
        pl.kernel,
        mesh=_mesh(),
        out_type=(
            jax.ShapeDtypeStruct((2 * E,), jnp.float32),
            jax.ShapeDtypeStruct((2 * E,), jnp.float32),
        ),
        scratch_types=[
            pltpu.VMEM((1024,), jnp.int32),
            pltpu.VMEM((1024,), jnp.int32),
            pltpu.VMEM((128, 128), jnp.float32),
            pltpu.VMEM((128, 128), jnp.float32),
            pltpu.VMEM((1024,), jnp.float32),
            pltpu.SemaphoreType.DMA,
        ],
        compiler_params=_params,
    )
    def score(u0, u1, i0, i1, ps_ref, pd_ref, ns_ref, nd_ref,
              pos_out, neg_out, srcv, dstv, urows, irows, scores, sem):
        core = lax.axis_index("c")
        sub = lax.axis_index("s")
        iota = lax.iota(jnp.int32, L)
        base = sub * 1024

        def gather_rows(emb, idx_ref, dst):
            pltpu.async_copy(emb.at[idx_ref], dst, sem).wait()

        def score_edges(src_hbm, dst_hbm, out_hbm):
            pltpu.sync_copy(src_hbm.at[pl.ds(base, 1024)], srcv)
            pltpu.sync_copy(dst_hbm.at[pl.ds(base, 1024)], dstv)
            for ch in range(8):
                si = srcv.at[pl.ds(ch * 128, 128)]
                di = dstv.at[pl.ds(ch * 128, 128)]

                @pl.when(core == 0)
                def _g0():
                    gather_rows(u0, si, urows)
                    gather_rows(i0, di, irows)

                @pl.when(core == 1)
                def _g1():
                    gather_rows(u1, si, urows)
                    gather_rows(i1, di, irows)
                for g in range(8):
                    rows = g * L + iota
                    acc = jnp.zeros((L,), jnp.float32)
                    for d in range(L):
                        dv = jnp.full((L,), d, jnp.int32)
                        acc = acc + (plsc.load_gather(urows, [rows, dv])
                                     * plsc.load_gather(irows, [rows, dv]))
                    scores[pl.ds(ch * 128 + g * L, L)] = acc
            pltpu.sync_copy(scores, out_hbm.at[pl.ds(core * E + base, 1024)])

        score_edges(ps_ref, pd_ref, pos_out)
        score_edges(ns_ref, nd_ref, neg_out)

    return score


def kernel(user_table, item_table, user_idx, item_idx, pos_src, pos_dst,
           neg_src, neg_dst):
    extract = _build_extract()
    score = _build_score()
    ttu = user_table.T.reshape(4, 8, V)
    tti = item_table.T.reshape(4, 8, V)
    tlu = user_table[TAIL0:].T.reshape(-1)
    tli = item_table[TAIL0:].T.reshape(-1)
    u0, u1, i0, i1 = extract(
        ttu, tti, tlu, tli,
        user_idx.astype(jnp.int32), item_idx.astype(jnp.int32))
    pos_parts, neg_parts = score(
        u0, u1, i0, i1,
        pos_src.astype(jnp.int32), pos_dst.astype(jnp.int32),
        neg_src.astype(jnp.int32), neg_dst.astype(jnp.int32))
    pos = (pos_parts[:E] + pos_parts[E:]).reshape(E, 1)
    neg = (neg_parts[:E] + neg_parts[E:]).reshape(E, 1)
    return pos, neg


# speedup vs baseline: 6.1222x; 6.1222x over previous
import functools

import jax
import jax.numpy as jnp
from jax import lax
from jax.experimental import pallas as pl
from jax.experimental.pallas import tpu as pltpu
from jax.experimental.pallas import tpu_sc as plsc

N_NODES = 16384
N_EDGES = 16384
EMBED_DIM = 32
LANES = 16


@functools.lru_cache(maxsize=None)
def _build_mf_kernel():
    info = plsc.get_sparse_core_info()
    nc, ns = info.num_cores, info.num_subcores
    nw = nc * ns
    epw = N_EDGES // nw
    n_chunks = epw // 128

    mesh = plsc.VectorSubcoreMesh(core_axis_name="c", subcore_axis_name="s")

    @functools.partial(
        pl.kernel,
        mesh=mesh,
        out_type=(
            jax.ShapeDtypeStruct((N_EDGES,), jnp.float32),
            jax.ShapeDtypeStruct((N_EDGES,), jnp.float32),
        ),
        scratch_types=[
            pltpu.VMEM((N_NODES,), jnp.int32),
            pltpu.VMEM((N_NODES,), jnp.int32),
            pltpu.VMEM((epw,), jnp.int32),
            pltpu.VMEM((epw,), jnp.int32),
            pltpu.VMEM((n_chunks, 128), jnp.int32),
            pltpu.VMEM((n_chunks, 128), jnp.int32),
            pltpu.VMEM((epw, EMBED_DIM), jnp.float32),
            pltpu.VMEM((epw, EMBED_DIM), jnp.float32),
            pltpu.VMEM((epw,), jnp.float32),
            pltpu.SemaphoreType.DMA,
        ],
        compiler_params=pltpu.CompilerParams(
            needs_layout_passes=False, use_tc_tiling_on_sc=False),
    )
    def mf(ut_ref, it_ref, uidx_ref, iidx_ref, ps_ref, pd_ref, nsrc_ref,
           nd_ref, pos_out, neg_out, uidx_v, iidx_v, src_v, dst_v,
           gid_u, gid_i, urows, irows, scores_v, sem):
        wid = lax.axis_index("s") * nc + lax.axis_index("c")
        base = wid * epw
        iota16 = lax.iota(jnp.int32, LANES)

        pltpu.sync_copy(uidx_ref, uidx_v)
        pltpu.sync_copy(iidx_ref, iidx_v)

        for s_hbm, d_hbm, o_hbm in ((ps_ref, pd_ref, pos_out),
                                    (nsrc_ref, nd_ref, neg_out)):
            pltpu.sync_copy(s_hbm.at[pl.ds(base, epw)], src_v)
            pltpu.sync_copy(d_hbm.at[pl.ds(base, epw)], dst_v)

            for c in range(epw // LANES):
                s = src_v[pl.ds(c * LANES, LANES)]
                d = dst_v[pl.ds(c * LANES, LANES)]
                gu = plsc.load_gather(uidx_v, [s])
                gi = plsc.load_gather(iidx_v, [d])
                r, off = divmod(c * LANES, 128)
                gid_u[r, pl.ds(off, LANES)] = gu
                gid_i[r, pl.ds(off, LANES)] = gi

            copies = []
            for j in range(n_chunks):
                copies.append(pltpu.async_copy(
                    ut_ref.at[gid_u.at[j]],
                    urows.at[pl.ds(j * 128, 128)], sem))
                copies.append(pltpu.async_copy(
                    it_ref.at[gid_i.at[j]],
                    irows.at[pl.ds(j * 128, 128)], sem))
            for cp in copies:
                cp.wait()

            def group_body(g, carry):
                row = g * LANES + iota16
                acc = jnp.zeros((LANES,), jnp.float32)
                for dcol in range(EMBED_DIM):
                    col = jnp.full((LANES,), dcol, jnp.int32)
                    ucol = plsc.load_gather(urows, [row, col])
                    icol = plsc.load_gather(irows, [row, col])
                    acc = acc + ucol * icol
                plsc.store_scatter(scores_v, [row], acc)
                return carry

            lax.fori_loop(0, epw // LANES, group_body, 0)
            pltpu.sync_copy(scores_v, o_hbm.at[pl.ds(base, epw)])

    return mf


def kernel(user_table, item_table, user_idx, item_idx, pos_src, pos_dst,
           neg_src, neg_dst):
    mf = _build_mf_kernel()
    pos, neg = mf(
        user_table, item_table,
        user_idx.astype(jnp.int32), item_idx.astype(jnp.int32),
        pos_src.astype(jnp.int32), pos_dst.astype(jnp.int32),
        neg_src.astype(jnp.int32), neg_dst.astype(jnp.int32),
    )
    return pos.reshape(N_EDGES, 1), neg.reshape(N_EDGES, 1)
